# baseline (speedup 1.0000x reference)
import functools

import jax
import jax.numpy as jnp
from jax.experimental import pallas as pl

_DECAY = 0.1
_EPS = 1e-8
_LN_EPS = 1e-5
_K = 4


def _body(*refs, F, T, D):
    h_refs = refs[:_K]
    lens_ref, scale_ref, bias_ref, out_ref = refs[_K:]
    lens_f = lens_ref[...].astype(jnp.float32)
    scale = scale_ref[...].reshape(1, 1, D)
    bias = bias_ref[...].reshape(1, 1, D)
    r = jnp.exp(jnp.float32(_DECAY))
    for k in range(_K):
        lens4 = lens_f[k].reshape(1, F, 1, 1)
        t_idx = jax.lax.broadcasted_iota(jnp.int32, (1, F, T, 1), 2)
        w = jnp.exp(-_DECAY * ((T - 1) - t_idx).astype(jnp.float32))
        w = jnp.where(t_idx.astype(jnp.float32) < lens4, w, 0.0)
        e = jnp.sum(h_refs[k][...] * w, axis=2)
        lens3 = lens4[:, :, :, 0]
        wsum = jnp.exp(-_DECAY * (T - 1)) * (jnp.exp(_DECAY * lens3) - 1.0) / (r - 1.0)
        wsum = jnp.maximum(wsum, _EPS)
        e = e / wsum
        mu = jnp.mean(e, axis=2, keepdims=True)
        var = jnp.mean((e - mu) ** 2, axis=2, keepdims=True)
        e_ln = (e - mu) * jax.lax.rsqrt(var + _LN_EPS) * scale + bias
        out_ref[k : k + 1] = jnp.where(lens3 >= 1.0, e_ln, e)


def kernel(H, valid_lens, ln_scale, ln_bias):
    B, F, T, D = H.shape
    lens2 = valid_lens.astype(jnp.int32).reshape(B, 1, F)
    scale2 = ln_scale.reshape(1, D)
    bias2 = ln_bias.reshape(1, D)

    def h_spec(k):
        return pl.BlockSpec((1, F, T, D), lambda i, k=k: (_K * i + k, 0, 0, 0))

    out = pl.pallas_call(
        functools.partial(_body, F=F, T=T, D=D),
        grid=(B // _K,),
        in_specs=[h_spec(k) for k in range(_K)]
        + [
            pl.BlockSpec((_K, 1, F), lambda i: (i, 0, 0)),
            pl.BlockSpec((1, D), lambda i: (0, 0)),
            pl.BlockSpec((1, D), lambda i: (0, 0)),
        ],
        out_specs=pl.BlockSpec((_K, F, D), lambda i: (i, 0, 0)),
        out_shape=jax.ShapeDtypeStruct((B, F, D), jnp.float32),
    )(H, H, H, H, lens2, scale2, bias2)
    return out
